# baseline (speedup 1.0000x reference)
import functools

import jax
import jax.numpy as jnp
from jax import lax
from jax.experimental import pallas as pl
from jax.experimental.pallas import tpu as pltpu
from jax.experimental.pallas import tpu_sc as plsc

N = 10000
E = 320000
D_IN = 128
HID = 128
HEADS = 8
DIM = HID // HEADS
OUT = 3

_BN = 1000


def _proj_body(x_ref, w_ref, b_ref, a_ref, h_ref, ah_ref):
    h = jnp.dot(x_ref[...], w_ref[...], preferred_element_type=jnp.float32)
    h = h + b_ref[...]
    h_ref[...] = h
    ah_ref[...] = jnp.dot(h, a_ref[...], preferred_element_type=jnp.float32)


def _project(x, W, b2, A_cat):
    grid = (N // _BN,)
    return pl.pallas_call(
        _proj_body,
        grid=grid,
        in_specs=[
            pl.BlockSpec((_BN, D_IN), lambda i: (i, 0)),
            pl.BlockSpec((D_IN, HID), lambda i: (0, 0)),
            pl.BlockSpec((1, HID), lambda i: (0, 0)),
            pl.BlockSpec((HID, 4 * HEADS), lambda i: (0, 0)),
        ],
        out_specs=[
            pl.BlockSpec((_BN, HID), lambda i: (i, 0)),
            pl.BlockSpec((_BN, 4 * HEADS), lambda i: (i, 0)),
        ],
        out_shape=[
            jax.ShapeDtypeStruct((N, HID), jnp.float32),
            jax.ShapeDtypeStruct((N, 4 * HEADS), jnp.float32),
        ],
    )(x, W, b2, A_cat)


def _blockdiag(att):
    eye = jnp.eye(HEADS, dtype=att.dtype)
    return (att[:, :, None] * eye[:, None, :]).reshape(HID, HEADS)


_NC = 2
_NS = 16
_NPAD = 10240
_RT = _NPAD // _NS
_ECH = 64
_SUP = 8
_NSUPER = 40
_EPT = _NSUPER * _SUP * _ECH
_EPAD = _NS * _EPT
_NB = 64


def _sc_edge_pass(src3, dst3, a_src2, a_dst2, h_pad):
    mesh = plsc.VectorSubcoreMesh(
        core_axis_name="c", subcore_axis_name="s",
        num_cores=_NC, num_subcores=_NS)

    @functools.partial(
        pl.kernel,
        compiler_params=pltpu.CompilerParams(use_tc_tiling_on_sc=False),
        out_type=jax.ShapeDtypeStruct((_NC, _NPAD, HID), jnp.float32),
        mesh=mesh,
        scratch_types=[
            pltpu.VMEM_SHARED((_NPAD, HID), jnp.float32),
            pltpu.VMEM_SHARED((_NPAD, 16), jnp.float32),
            pltpu.VMEM((_SUP, _ECH), jnp.int32),
            pltpu.VMEM((_SUP, _ECH), jnp.int32),
            [pltpu.VMEM((_ECH, 16), jnp.float32)] * 2,
            [pltpu.VMEM((_ECH, 16), jnp.float32)] * 2,
            [pltpu.VMEM((_ECH, 16), jnp.float32)] * 2,
            [pltpu.VMEM((_ECH, HID), jnp.float32)] * 2,
            pltpu.VMEM((_NB, 16), jnp.float32),
            [pltpu.SemaphoreType.DMA] * 2,
            [pltpu.SemaphoreType.DMA] * 2,
            [pltpu.SemaphoreType.DMA] * 2,
            [pltpu.SemaphoreType.DMA] * 2,
        ],
    )
    def k(src_hbm, dst_hbm, asrc_hbm, adst_hbm, h_hbm, outs_hbm,
          out_acc, s_acc, src_sc, dst_sc, asrc, adst, ebuf, hg, sbuf,
          sem_ga, sem_gh, sem_se, sem_so):
        c = lax.axis_index("c")
        sid = lax.axis_index("s")
        zero16 = jnp.zeros((16,), jnp.float32)

        def zrow(i, _):
            for j in range(HID // 16):
                hg[0][i, pl.ds(j * 16, 16)] = zero16
            sbuf[i, :] = zero16
            return 0
        lax.fori_loop(0, _NB, zrow, 0)

        def zinit(kk, _):
            r0 = sid * _RT + kk * _NB
            pltpu.sync_copy(hg[0], out_acc.at[pl.ds(r0, _NB)])
            pltpu.sync_copy(sbuf, s_acc.at[pl.ds(r0, _NB)])
            return 0
        lax.fori_loop(0, _RT // _NB, zinit, 0)
        plsc.subcore_barrier()

        def issue_gathers(j, descs):
            b = j % 2
            sv = src_sc.at[j]
            dv = dst_sc.at[j]
            descs['ga_s', j] = pltpu.async_copy(
                asrc_hbm.at[c].at[sv], asrc[b], sem_ga[b])
            descs['ga_d', j] = pltpu.async_copy(
                adst_hbm.at[c].at[dv], adst[b], sem_ga[b])
            descs['gh', j] = pltpu.async_copy(h_hbm.at[sv], hg[b], sem_gh[b])

        def drain_scatters(b):
            pltpu.make_async_copy(
                asrc_hbm.at[0, pl.ds(0, _ECH)], ebuf[b], sem_se[b]).wait()
            pltpu.make_async_copy(
                h_hbm.at[pl.ds(0, _ECH)], hg[b], sem_so[b]).wait()

        def super_chunk(t, _):
            @pl.when(t > 0)
            def _():
                drain_scatters(0)
                drain_scatters(1)
            row0 = sid * (_NSUPER * _SUP) + t * _SUP
            pltpu.sync_copy(src_hbm.at[c, pl.ds(row0, _SUP)], src_sc)
            pltpu.sync_copy(dst_hbm.at[c, pl.ds(row0, _SUP)], dst_sc)
            descs = {}
            issue_gathers(0, descs)
            for j in range(_SUP):
                b = j % 2
                if j < _SUP - 1:
                    if j >= 1:
                        descs['se', j - 1].wait()
                        descs['so', j - 1].wait()
                    issue_gathers(j + 1, descs)
                descs['ga_s', j].wait()
                descs['ga_d', j].wait()
                descs['gh', j].wait()

                def edge(i, _):
                    a = asrc[b][i, :] + adst[b][i, :]
                    a = jnp.where(a >= 0.0, a, 0.2 * a)
                    ev = jnp.exp(a)
                    ebuf[b][i, :] = ev
                    for hh in range(HEADS):
                        seg = hg[b][i, pl.ds(hh * 16, 16)]
                        hg[b][i, pl.ds(hh * 16, 16)] = seg * ev[hh]
                    return 0
                lax.fori_loop(0, _ECH, edge, 0, unroll=4)
                descs['se', j] = pltpu.async_copy(
                    ebuf[b], s_acc.at[dst_sc.at[j]], sem_se[b], add=True)
                descs['so', j] = pltpu.async_copy(
                    hg[b], out_acc.at[dst_sc.at[j]], sem_so[b], add=True)
            return 0
        lax.fori_loop(0, _NSUPER, super_chunk, 0)
        drain_scatters(0)
        drain_scatters(1)
        plsc.subcore_barrier()

        nbase = sid * _RT

        def p3(kk, _):
            r0 = nbase + kk * _NB
            pltpu.sync_copy(out_acc.at[pl.ds(r0, _NB)], hg[0])
            pltpu.sync_copy(s_acc.at[pl.ds(r0, _NB)], sbuf)

            def row(i, _):
                rv = jnp.float32(1.0) / (sbuf[i, :] + 1e-16)
                for hh in range(HEADS):
                    v = hg[0][i, pl.ds(hh * 16, 16)] * rv[hh]
                    hg[0][i, pl.ds(hh * 16, 16)] = jnp.maximum(v, 0.0)
                return 0
            lax.fori_loop(0, _NB, row, 0, unroll=4)
            pltpu.sync_copy(hg[0], outs_hbm.at[c, pl.ds(r0, _NB)])
            return 0
        lax.fori_loop(0, _RT // _NB, p3, 0)

    return k(src3, dst3, a_src2, a_dst2, h_pad)


def _sem_score_body(o_ref, kw_ref, kb_ref, q_ref, acc_ref):
    p = pl.program_id(0)
    i = pl.program_id(1)
    k = jnp.dot(o_ref[...], kw_ref[...], preferred_element_type=jnp.float32)
    k = jnp.tanh(k + kb_ref[...])
    u = jnp.sum(k * q_ref[...])

    @pl.when(jnp.logical_and(p == 0, i == 0))
    def _():
        acc_ref[...] = jnp.zeros_like(acc_ref)

    onehot = (jax.lax.broadcasted_iota(jnp.int32, (1, 2), 1) == p)
    acc_ref[...] = acc_ref[...] + jnp.where(onehot, u, 0.0)


def _sem_scores(outs, K_w, K_b2, q2):
    grid = (2, N // _BN)
    return pl.pallas_call(
        _sem_score_body,
        grid=grid,
        in_specs=[
            pl.BlockSpec((1, _BN, HID), lambda p, i: (p, i, 0)),
            pl.BlockSpec((HID, HID), lambda p, i: (0, 0)),
            pl.BlockSpec((1, HID), lambda p, i: (0, 0)),
            pl.BlockSpec((1, HID), lambda p, i: (0, 0)),
        ],
        out_specs=pl.BlockSpec((1, 2), lambda p, i: (0, 0)),
        out_shape=jax.ShapeDtypeStruct((1, 2), jnp.float32),
    )(outs, K_w, K_b2, q2)


def _final_body(score_ref, o0_ref, o1_ref, lw_ref, lb_ref, out_ref):
    s0 = score_ref[0, 0]
    s1 = score_ref[0, 1]
    m = jnp.maximum(s0, s1)
    e0 = jnp.exp(s0 - m)
    e1 = jnp.exp(s1 - m)
    a0 = e0 / (e0 + e1)
    a1 = e1 / (e0 + e1)
    sem = a0 * o0_ref[...] + a1 * o1_ref[...]
    out_ref[...] = jnp.dot(sem, lw_ref[...], preferred_element_type=jnp.float32) + lb_ref[...]


def _final(score, o0, o1, L_w, L_b2):
    grid = (N // _BN,)
    return pl.pallas_call(
        _final_body,
        grid=grid,
        in_specs=[
            pl.BlockSpec(memory_space=pltpu.SMEM),
            pl.BlockSpec((_BN, HID), lambda i: (i, 0)),
            pl.BlockSpec((_BN, HID), lambda i: (i, 0)),
            pl.BlockSpec((HID, OUT), lambda i: (0, 0)),
            pl.BlockSpec((1, OUT), lambda i: (0, 0)),
        ],
        out_specs=pl.BlockSpec((_BN, OUT), lambda i: (i, 0)),
        out_shape=jax.ShapeDtypeStruct((N, OUT), jnp.float32),
    )(score, o0, o1, L_w, L_b2)


def kernel(x_movie, edge_index_mp0, edge_index_mp1, W_proj, b_proj,
           att_src_0, att_dst_0, att_src_1, att_dst_1,
           K_w, K_b, q, L_w, L_b):
    A_cat = jnp.concatenate(
        [_blockdiag(att_src_0), _blockdiag(att_dst_0),
         _blockdiag(att_src_1), _blockdiag(att_dst_1)], axis=1)
    h, ah = _project(x_movie, W_proj, b_proj.reshape(1, HID), A_cat)

    pad_e = _EPAD - E
    src3 = jnp.concatenate(
        [jnp.stack([edge_index_mp0[0], edge_index_mp1[0]]),
         jnp.full((2, pad_e), N, jnp.int32)], axis=1).reshape(2, _EPAD // _ECH, _ECH)
    dst3 = jnp.concatenate(
        [jnp.stack([edge_index_mp0[1], edge_index_mp1[1]]),
         jnp.full((2, pad_e), N, jnp.int32)], axis=1).reshape(2, _EPAD // _ECH, _ECH)
    z8 = jnp.zeros((N, 8), jnp.float32)
    a_src2 = jnp.stack([jnp.concatenate([ah[:, 0:8], z8], 1),
                        jnp.concatenate([ah[:, 16:24], z8], 1)])
    a_dst2 = jnp.stack([jnp.concatenate([ah[:, 8:16], z8], 1),
                        jnp.concatenate([ah[:, 24:32], z8], 1)])
    a_src2 = jnp.pad(a_src2, ((0, 0), (0, _NPAD - N), (0, 0)))
    a_dst2 = jnp.pad(a_dst2, ((0, 0), (0, _NPAD - N), (0, 0)))
    h_pad = jnp.pad(h, ((0, _NPAD - N), (0, 0)))

    outs_pad = _sc_edge_pass(src3, dst3, a_src2, a_dst2, h_pad)
    o0 = outs_pad[0, :N]
    o1 = outs_pad[1, :N]
    outs = jnp.stack([o0, o1], axis=0)
    score = _sem_scores(outs, K_w, K_b.reshape(1, HID), q.reshape(1, HID))
    score = score / jnp.float32(N)
    return _final(score, o0, o1, L_w, L_b.reshape(1, OUT))


# speedup vs baseline: 1.7043x; 1.7043x over previous
import functools

import jax
import jax.numpy as jnp
from jax import lax
from jax.experimental import pallas as pl
from jax.experimental.pallas import tpu as pltpu
from jax.experimental.pallas import tpu_sc as plsc

N = 10000
E = 320000
D_IN = 128
HID = 128
HEADS = 8
DIM = HID // HEADS
OUT = 3

_BN = 1000


def _proj_body(x_ref, w_ref, b_ref, a_ref, h_ref, ah_ref):
    h = jnp.dot(x_ref[...], w_ref[...], preferred_element_type=jnp.float32)
    h = h + b_ref[...]
    h_ref[...] = h
    ah_ref[...] = jnp.dot(h, a_ref[...], preferred_element_type=jnp.float32)


def _project(x, W, b2, A_cat):
    grid = (N // _BN,)
    return pl.pallas_call(
        _proj_body,
        grid=grid,
        in_specs=[
            pl.BlockSpec((_BN, D_IN), lambda i: (i, 0)),
            pl.BlockSpec((D_IN, HID), lambda i: (0, 0)),
            pl.BlockSpec((1, HID), lambda i: (0, 0)),
            pl.BlockSpec((HID, 4 * HEADS), lambda i: (0, 0)),
        ],
        out_specs=[
            pl.BlockSpec((_BN, HID), lambda i: (i, 0)),
            pl.BlockSpec((_BN, 4 * HEADS), lambda i: (i, 0)),
        ],
        out_shape=[
            jax.ShapeDtypeStruct((N, HID), jnp.float32),
            jax.ShapeDtypeStruct((N, 4 * HEADS), jnp.float32),
        ],
    )(x, W, b2, A_cat)


def _blockdiag(att):
    eye = jnp.eye(HEADS, dtype=att.dtype)
    return (att[:, :, None] * eye[:, None, :]).reshape(HID, HEADS)


_NC = 2
_NS = 16
_NPAD = 10112
_RT = _NPAD // _NS
_NB = 79
_ECH = 112
_SUP = 4
_NSUPER = 45
_NCHT = _NSUPER * _SUP
_EPT = _NCHT * _ECH
_EPAD = _NS * _EPT
_W = HID + 16


def _sc_edge_pass(ei4, hs2, adst2):
    mesh = plsc.VectorSubcoreMesh(
        core_axis_name="c", subcore_axis_name="s",
        num_cores=_NC, num_subcores=_NS)

    @functools.partial(
        pl.kernel,
        compiler_params=pltpu.CompilerParams(use_tc_tiling_on_sc=False),
        out_type=jax.ShapeDtypeStruct((_NC, _NPAD, _W), jnp.float32),
        mesh=mesh,
        scratch_types=[
            pltpu.VMEM_SHARED((_NPAD, _W), jnp.float32),
            pltpu.VMEM((_SUP, 2, _ECH), jnp.int32),
            [pltpu.VMEM((_ECH, _W), jnp.float32)] * 2,
            [pltpu.VMEM((_ECH, 16), jnp.float32)] * 2,
            [pltpu.SemaphoreType.DMA] * 2,
            [pltpu.SemaphoreType.DMA] * 2,
            [pltpu.SemaphoreType.DMA] * 2,
        ],
    )
    def k(ei_hbm, hs_hbm, adst_hbm, outs_hbm,
          acc, ei_sc, hs, adst, sem_ga, sem_gh, sem_so):
        c = lax.axis_index("c")
        sid = lax.axis_index("s")
        zero16 = jnp.zeros((16,), jnp.float32)

        def zrow(i, _):
            for j in range(_W // 16):
                hs[0][i, pl.ds(j * 16, 16)] = zero16
            return 0
        lax.fori_loop(0, _NB, zrow, 0)

        def zinit(kk, _):
            r0 = sid * _RT + kk * _NB
            pltpu.sync_copy(hs[0].at[pl.ds(0, _NB)], acc.at[pl.ds(r0, _NB)])
            return 0
        lax.fori_loop(0, _RT // _NB, zinit, 0)
        plsc.subcore_barrier()

        def issue_gathers(j, descs):
            b = j % 2
            descs['gh', j] = pltpu.async_copy(
                hs_hbm.at[c].at[ei_sc.at[j, 0]], hs[b], sem_gh[b])
            descs['ga', j] = pltpu.async_copy(
                adst_hbm.at[c].at[ei_sc.at[j, 1]], adst[b], sem_ga[b])

        def drain_scatter(b):
            pltpu.make_async_copy(
                hs_hbm.at[0, pl.ds(0, _ECH)], hs[b], sem_so[b]).wait()

        def super_chunk(t, _):
            @pl.when(t > 0)
            def _():
                drain_scatter(0)
                drain_scatter(1)
            row0 = sid * _NCHT + t * _SUP
            pltpu.sync_copy(ei_hbm.at[c, pl.ds(row0, _SUP)], ei_sc)
            descs = {}
            issue_gathers(0, descs)
            for j in range(_SUP):
                b = j % 2
                if j < _SUP - 1:
                    if j >= 1:
                        descs['so', j - 1].wait()
                    issue_gathers(j + 1, descs)
                descs['gh', j].wait()
                descs['ga', j].wait()

                def edge(i, _):
                    a = hs[b][i, pl.ds(HID, 16)] + adst[b][i, :]
                    a = jnp.where(a >= 0.0, a, 0.2 * a)
                    ev = jnp.exp(a)
                    hs[b][i, pl.ds(HID, 16)] = ev
                    for hh in range(HEADS):
                        seg = hs[b][i, pl.ds(hh * 16, 16)]
                        hs[b][i, pl.ds(hh * 16, 16)] = seg * ev[hh]
                    return 0
                lax.fori_loop(0, _ECH, edge, 0)
                descs['so', j] = pltpu.async_copy(
                    hs[b], acc.at[ei_sc.at[j, 1]], sem_so[b], add=True)
            return 0
        lax.fori_loop(0, _NSUPER, super_chunk, 0)
        drain_scatter(0)
        drain_scatter(1)
        plsc.subcore_barrier()

        nbase = sid * _RT

        def p3(kk, _):
            r0 = nbase + kk * _NB
            pltpu.sync_copy(acc.at[pl.ds(r0, _NB)], hs[0].at[pl.ds(0, _NB)])

            def row(i, _):
                rv = jnp.float32(1.0) / (hs[0][i, pl.ds(HID, 16)] + 1e-16)
                for hh in range(HEADS):
                    v = hs[0][i, pl.ds(hh * 16, 16)] * rv[hh]
                    hs[0][i, pl.ds(hh * 16, 16)] = jnp.maximum(v, 0.0)
                return 0
            lax.fori_loop(0, _NB, row, 0)
            pltpu.sync_copy(hs[0].at[pl.ds(0, _NB)], outs_hbm.at[c, pl.ds(r0, _NB)])
            return 0
        lax.fori_loop(0, _RT // _NB, p3, 0)

    return k(ei4, hs2, adst2)


def _sem_score_body(o_ref, kw_ref, kb_ref, q_ref, acc_ref):
    p = pl.program_id(0)
    i = pl.program_id(1)
    k = jnp.dot(o_ref[...], kw_ref[...], preferred_element_type=jnp.float32)
    k = jnp.tanh(k + kb_ref[...])
    u = jnp.sum(k * q_ref[...])

    @pl.when(jnp.logical_and(p == 0, i == 0))
    def _():
        acc_ref[...] = jnp.zeros_like(acc_ref)

    onehot = (jax.lax.broadcasted_iota(jnp.int32, (1, 2), 1) == p)
    acc_ref[...] = acc_ref[...] + jnp.where(onehot, u, 0.0)


def _sem_scores(outs, K_w, K_b2, q2):
    grid = (2, N // _BN)
    return pl.pallas_call(
        _sem_score_body,
        grid=grid,
        in_specs=[
            pl.BlockSpec((1, _BN, HID), lambda p, i: (p, i, 0)),
            pl.BlockSpec((HID, HID), lambda p, i: (0, 0)),
            pl.BlockSpec((1, HID), lambda p, i: (0, 0)),
            pl.BlockSpec((1, HID), lambda p, i: (0, 0)),
        ],
        out_specs=pl.BlockSpec((1, 2), lambda p, i: (0, 0)),
        out_shape=jax.ShapeDtypeStruct((1, 2), jnp.float32),
    )(outs, K_w, K_b2, q2)


def _final_body(score_ref, o0_ref, o1_ref, lw_ref, lb_ref, out_ref):
    s0 = score_ref[0, 0]
    s1 = score_ref[0, 1]
    m = jnp.maximum(s0, s1)
    e0 = jnp.exp(s0 - m)
    e1 = jnp.exp(s1 - m)
    a0 = e0 / (e0 + e1)
    a1 = e1 / (e0 + e1)
    sem = a0 * o0_ref[...] + a1 * o1_ref[...]
    out_ref[...] = jnp.dot(sem, lw_ref[...], preferred_element_type=jnp.float32) + lb_ref[...]


def _final(score, o0, o1, L_w, L_b2):
    grid = (N // _BN,)
    return pl.pallas_call(
        _final_body,
        grid=grid,
        in_specs=[
            pl.BlockSpec(memory_space=pltpu.SMEM),
            pl.BlockSpec((_BN, HID), lambda i: (i, 0)),
            pl.BlockSpec((_BN, HID), lambda i: (i, 0)),
            pl.BlockSpec((HID, OUT), lambda i: (0, 0)),
            pl.BlockSpec((1, OUT), lambda i: (0, 0)),
        ],
        out_specs=pl.BlockSpec((_BN, OUT), lambda i: (i, 0)),
        out_shape=jax.ShapeDtypeStruct((N, OUT), jnp.float32),
    )(score, o0, o1, L_w, L_b2)


def kernel(x_movie, edge_index_mp0, edge_index_mp1, W_proj, b_proj,
           att_src_0, att_dst_0, att_src_1, att_dst_1,
           K_w, K_b, q, L_w, L_b):
    A_cat = jnp.concatenate(
        [_blockdiag(att_src_0), _blockdiag(att_dst_0),
         _blockdiag(att_src_1), _blockdiag(att_dst_1)], axis=1)
    h, ah = _project(x_movie, W_proj, b_proj.reshape(1, HID), A_cat)

    pad_e = _EPAD - E
    src2 = jnp.concatenate(
        [jnp.stack([edge_index_mp0[0], edge_index_mp1[0]]),
         jnp.full((2, pad_e), N, jnp.int32)], axis=1)
    dst2 = jnp.concatenate(
        [jnp.stack([edge_index_mp0[1], edge_index_mp1[1]]),
         jnp.full((2, pad_e), N, jnp.int32)], axis=1)
    ei4 = jnp.stack([src2.reshape(2, _NCHT * _NS, _ECH),
                     dst2.reshape(2, _NCHT * _NS, _ECH)], axis=2)
    z8 = jnp.zeros((N, 8), jnp.float32)
    hs2 = jnp.stack([
        jnp.concatenate([h, ah[:, 0:8], z8], 1),
        jnp.concatenate([h, ah[:, 16:24], z8], 1)])
    a_dst2 = jnp.stack([jnp.concatenate([ah[:, 8:16], z8], 1),
                        jnp.concatenate([ah[:, 24:32], z8], 1)])
    hs2 = jnp.pad(hs2, ((0, 0), (0, _NPAD - N), (0, 0)))
    a_dst2 = jnp.pad(a_dst2, ((0, 0), (0, _NPAD - N), (0, 0)))

    outs_pad = _sc_edge_pass(ei4, hs2, a_dst2)
    o0 = outs_pad[0, :N, :HID]
    o1 = outs_pad[1, :N, :HID]
    outs = jnp.stack([o0, o1], axis=0)
    score = _sem_scores(outs, K_w, K_b.reshape(1, HID), q.reshape(1, HID))
    score = score / jnp.float32(N)
    return _final(score, o0, o1, L_w, L_b.reshape(1, OUT))
